# baseline (speedup 1.0000x reference)
import functools

import jax
import jax.numpy as jnp
from jax import lax
from jax.experimental import pallas as pl
from jax.experimental.pallas import tpu as pltpu
from jax.experimental.pallas import tpu_sc as plsc

N = 10000
D = 128
E = 320000
G = 128

NC = 2
NS = 16
CHUNK = 128
EPW = 10240
NCHUNKS = EPW // CHUNK
E_PAD = NC * NS * EPW
NBUF = 2
IBLK = 8
NIB = 3
NBLK = NCHUNKS // IBLK
SH_ROWS = 10240
OUT_PER_S = SH_ROWS // NS

HIGH = jax.lax.Precision.HIGHEST


def _sc_agg(h, src, dst, zrows):
    mesh = plsc.VectorSubcoreMesh(core_axis_name="c", subcore_axis_name="s")

    @functools.partial(
        pl.kernel,
        out_type=jax.ShapeDtypeStruct((NC, SH_ROWS, D), jnp.float32),
        mesh=mesh,
        scratch_types=[
            pltpu.VMEM((NIB, IBLK, CHUNK), jnp.int32),
            pltpu.VMEM((NIB, IBLK, CHUNK), jnp.int32),
            pltpu.VMEM((NBUF, CHUNK, D), jnp.float32),
            pltpu.VMEM_SHARED((SH_ROWS, D), jnp.float32),
            [pltpu.SemaphoreType.DMA] * NBUF,
            [pltpu.SemaphoreType.DMA] * NBUF,
            pltpu.SemaphoreType.DMA,
        ],
    )
    def agg_kernel(h_hbm, src_hbm, dst_hbm, z_hbm, out_hbm, sblk, dblk, rows,
                   shared, sg, ss, si):
        c = lax.axis_index("c")
        s = lax.axis_index("s")

        zbase = s * OUT_PER_S
        pltpu.sync_copy(z_hbm, shared.at[pl.ds(zbase, OUT_PER_S)])
        plsc.subcore_barrier()

        def idx_issue(kb, p):
            pltpu.async_copy(src_hbm.at[c, s, pl.ds(kb * IBLK, IBLK)],
                             sblk.at[p], si)
            pltpu.async_copy(dst_hbm.at[c, s, pl.ds(kb * IBLK, IBLK)],
                             dblk.at[p], si)

        def idx_wait():
            pltpu.make_async_copy(src_hbm.at[c, s, pl.ds(0, IBLK)],
                                  sblk.at[0], si).wait()
            pltpu.make_async_copy(dst_hbm.at[c, s, pl.ds(0, IBLK)],
                                  dblk.at[0], si).wait()

        def g_issue(p, pos, b):
            pltpu.async_copy(h_hbm.at[sblk.at[p, pos]], rows.at[b], sg[b])

        def g_wait(b):
            pltpu.make_async_copy(h_hbm.at[sblk.at[0, 0]], rows.at[b],
                                  sg[b]).wait()

        def s_issue(p, pos, b):
            pltpu.async_copy(rows.at[b], shared.at[dblk.at[p, pos]], ss[b],
                             add=True)

        def s_wait(b):
            pltpu.make_async_copy(rows.at[b], shared.at[dblk.at[0, 0]],
                                  ss[b]).wait()

        idx_issue(0, 0)

        def block(kb, carry):
            p = lax.rem(kb, NIB)
            pm1 = lax.rem(kb + (NIB - 1), NIB)
            pnx = lax.rem(kb + 1, NIB)
            idx_wait()

            @pl.when(kb < NBLK - 1)
            def _():
                idx_issue(kb + 1, pnx)

            for pos in range(IBLK):
                b = pos % NBUF
                if pos >= NBUF:
                    s_wait(b)
                else:
                    @pl.when(kb >= 1)
                    def _():
                        s_wait(b)
                g_issue(p, pos, b)
                if pos >= 1:
                    g_wait(1 - b)
                    s_issue(p, pos - 1, 1 - b)
                else:
                    @pl.when(kb >= 1)
                    def _():
                        g_wait((IBLK - 1) % NBUF)
                        s_issue(pm1, IBLK - 1, (IBLK - 1) % NBUF)
            return carry
        lax.fori_loop(0, NBLK, block, 0)

        lastb = (IBLK - 1) % NBUF
        g_wait(lastb)
        s_issue((NBLK - 1) % NIB, IBLK - 1, lastb)
        for b in range(NBUF):
            s_wait(b)
        plsc.subcore_barrier()

        obase = s * OUT_PER_S
        pltpu.sync_copy(shared.at[pl.ds(obase, OUT_PER_S)],
                        out_hbm.at[c, pl.ds(obase, OUT_PER_S)])

    return agg_kernel(h, src, dst, zrows)


def _dense_one(h, agg, scale, Wa, ba, g1, be1, Wb, bb, act):
    def body(h_ref, a_ref, sc_ref, wa_ref, ba_ref, g_ref, be_ref, wb_ref,
             bb_ref, o_ref):
        z = sc_ref[0] * h_ref[...] + a_ref[0, :N] + a_ref[1, :N]
        z = jnp.dot(z, wa_ref[...], precision=HIGH,
                    preferred_element_type=jnp.float32) + ba_ref[...]
        m = jnp.mean(z, axis=0, keepdims=True)
        v = jnp.mean(jnp.square(z - m), axis=0, keepdims=True)
        z = g_ref[...] * (z - m) / jnp.sqrt(v + 1e-5) + be_ref[...]
        z = jnp.maximum(z, 0.0)
        z = jnp.dot(z, wb_ref[...], precision=HIGH,
                    preferred_element_type=jnp.float32) + bb_ref[...]
        if act:
            z = jnp.maximum(z, 0.0)
        o_ref[...] = z

    return pl.pallas_call(
        body,
        out_shape=jax.ShapeDtypeStruct((N, D), jnp.float32),
    )(h, agg, scale, Wa, ba, g1, be1, Wb, bb)


def _readout(h0, h1, ids_row, Wo1r, bo1r, Wo2, bo2r, Wo3p, bo3r):
    def body(h0_ref, h1_ref, ids_ref, w1_ref, b1_ref, w2_ref, b2_ref, w3_ref,
             b3_ref, o_ref):
        gi = lax.broadcasted_iota(jnp.int32, (G, N), 0)
        S = jnp.where(gi == ids_ref[...], 1.0, 0.0)
        hg0 = jnp.dot(S, h0_ref[...], precision=HIGH,
                      preferred_element_type=jnp.float32)
        hg1 = jnp.dot(S, h1_ref[...], precision=HIGH,
                      preferred_element_type=jnp.float32)
        o = (jnp.dot(hg0, w1_ref[0], precision=HIGH,
                     preferred_element_type=jnp.float32)
             + jnp.dot(hg1, w1_ref[1], precision=HIGH,
                       preferred_element_type=jnp.float32)
             + b1_ref[...])
        o = jnp.maximum(o, 0.0)
        o = jnp.dot(o, w2_ref[...], precision=HIGH,
                    preferred_element_type=jnp.float32) + b2_ref[...]
        o = jnp.maximum(o, 0.0)
        o = jnp.dot(o, w3_ref[...], precision=HIGH,
                    preferred_element_type=jnp.float32) + b3_ref[...]
        o_ref[...] = o

    return pl.pallas_call(
        body,
        out_shape=jax.ShapeDtypeStruct((G, D), jnp.float32),
    )(h0, h1, ids_row, Wo1r, bo1r, Wo2, bo2r, Wo3p, bo3r)


def kernel(x, edge_index_0, edge_index_1, node_graph_ids, eps, W_a, b_a, g, be,
           W_b, b_b, Wo1, bo1, Wo2, bo2, Wo3, bo3):
    pad = E_PAD - E

    pad_src = jnp.arange(pad, dtype=jnp.int32) * 37 % N
    pad_dst = N + (jnp.arange(pad, dtype=jnp.int32) % (SH_ROWS - N))

    def prep(a, fill):
        return jnp.concatenate([a, fill]).reshape(NC, NS, NCHUNKS, CHUNK)

    src0 = prep(edge_index_0[0], pad_src)
    src1 = prep(edge_index_1[0], pad_src)
    dst0 = prep(edge_index_0[1], pad_dst)
    dst1 = prep(edge_index_1[1], pad_dst)

    def sc(i):
        return (1.0 + eps[i]).reshape(1, 1)

    def brow(w, i):
        return w[i].reshape(1, D)

    zrows = jnp.zeros((OUT_PER_S, D), jnp.float32)

    a1_0 = _sc_agg(x, src0, dst0, zrows)
    src1b, _ = lax.optimization_barrier((src1, a1_0))
    a1_1 = _sc_agg(x, src1b, dst1, zrows)
    h1_0 = _dense_one(x, a1_0, sc(0), W_a[0], brow(b_a, 0), brow(g, 0),
                      brow(be, 0), W_b[0], brow(b_b, 0), act=True)
    h1_1 = _dense_one(x, a1_1, sc(2), W_a[2], brow(b_a, 2), brow(g, 2),
                      brow(be, 2), W_b[2], brow(b_b, 2), act=True)
    h12 = jnp.concatenate([h1_0, h1_1], axis=0)
    src0b, _ = lax.optimization_barrier((src0, a1_1))
    a2_0 = _sc_agg(h12, src0b, dst0, zrows)
    src1N = prep(edge_index_1[0] + N, pad_src + N)
    src1c, _ = lax.optimization_barrier((src1N, a2_0))
    a2_1 = _sc_agg(h12, src1c, dst1, zrows)
    h2_0 = _dense_one(h1_0, a2_0, sc(1), W_a[1], brow(b_a, 1), brow(g, 1),
                      brow(be, 1), W_b[1], brow(b_b, 1), act=False)
    h2_1 = _dense_one(h1_1, a2_1, sc(3), W_a[3], brow(b_a, 3), brow(g, 3),
                      brow(be, 3), W_b[3], brow(b_b, 3), act=False)

    ids_row = node_graph_ids.reshape(1, N)
    Wo1r = Wo1.reshape(2, D, D)
    Wo3p = jnp.pad(Wo3, ((0, 0), (0, D - 1)))
    bo3r = jnp.pad(bo3, (0, D - 1)).reshape(1, D)
    o = _readout(h2_0, h2_1, ids_row, Wo1r, bo1.reshape(1, D), Wo2,
                 bo2.reshape(1, D), Wo3p, bo3r)
    return o[:, :1]
